# baseline (speedup 1.0000x reference)
import functools

import jax
import jax.numpy as jnp
from jax import lax
from jax.experimental import pallas as pl
from jax.experimental.pallas import tpu as pltpu
from jax.experimental.pallas import tpu_sc as plsc

NUM_SEGS = 512
ROWS = 100000
DIM = 128
NQ = DIM // 16

TC_BLK = 664
TC_NBLK = 92
TC_ROWS = TC_BLK * TC_NBLK

NW = 32
CHUNK = 32
NCH = 38
RPW = NCH * CHUNK
SC_ROWS = NW * RPW
assert TC_ROWS + SC_ROWS == ROWS
NBUF = 4


def _sc_segment_sum(x, b3d):
    mesh = plsc.VectorSubcoreMesh(core_axis_name="c", subcore_axis_name="s")

    @functools.partial(
        pl.kernel,
        out_type=jax.ShapeDtypeStruct((2, NUM_SEGS, DIM), jnp.float32),
        mesh=mesh,
        compiler_params=pltpu.CompilerParams(needs_layout_passes=False),
        scratch_types=[
            pltpu.VMEM((NBUF, CHUNK, DIM), jnp.float32),
            pltpu.VMEM((NCH, CHUNK), jnp.int32),
            pltpu.VMEM((CHUNK, DIM), jnp.float32),
            pltpu.VMEM_SHARED((NUM_SEGS, DIM), jnp.float32),
            pltpu.SemaphoreType.DMA,
            pltpu.SemaphoreType.DMA,
            pltpu.SemaphoreType.DMA,
            pltpu.SemaphoreType.DMA,
            pltpu.SemaphoreType.DMA,
            pltpu.SemaphoreType.DMA,
            pltpu.SemaphoreType.DMA,
            pltpu.SemaphoreType.DMA,
        ],
    )
    def seg_kernel(x_hbm, b_hbm, out_hbm, xbuf, bbuf, zbuf, shacc,
                   sx0, sx1, sx2, sx3, ss0, ss1, ss2, ss3):
        c = lax.axis_index("c")
        s = lax.axis_index("s")
        wid = s * 2 + c
        row0 = TC_ROWS + wid * RPW
        sx = [sx0, sx1, sx2, sx3]
        ss = [ss0, ss1, ss2, ss3]

        def x_copy(k, b):
            return pltpu.make_async_copy(
                x_hbm.at[pl.ds(row0 + k * CHUNK, CHUNK), :],
                xbuf.at[b], sx[b])

        def sc_copy(k, b):
            return pltpu.make_async_copy(
                xbuf.at[b], shacc.at[bbuf.at[k]], ss[b])

        for b in range(2):
            x_copy(b, b).start()

        pltpu.sync_copy(b_hbm.at[wid], bbuf)

        zeros16 = jnp.zeros((16,), jnp.float32)

        def _zrow(r, carry):
            for q in range(NQ):
                zbuf[r, pl.ds(16 * q, 16)] = zeros16
            return carry

        lax.fori_loop(0, CHUNK, _zrow, 0)
        pltpu.sync_copy(zbuf, shacc.at[pl.ds(s * 32, 32), :])
        plsc.subcore_barrier()

        def _chunk_body(i, carry):
            for b in range(NBUF):
                k = NBUF * i + b

                @pl.when(k < NCH)
                def _work():
                    x_copy(k, b).wait()
                    sc_copy(k, b).start(add=True)

                if b < 2:
                    @pl.when(jnp.logical_and(k >= 2, k - 2 < NCH))
                    def _drain():
                        sc_copy(k - 2, (b + 2) % NBUF).wait()
                else:
                    @pl.when(k - 2 < NCH)
                    def _drain2():
                        sc_copy(k - 2, (b + 2) % NBUF).wait()

                @pl.when(k + 2 < NCH)
                def _prefetch():
                    x_copy(k + 2, (b + 2) % NBUF).start()

            return carry

        lax.fori_loop(0, (NCH + 2 + NBUF - 1) // NBUF, _chunk_body, 0)
        plsc.subcore_barrier()

        pltpu.sync_copy(shacc.at[pl.ds(s * 32, 32), :],
                        out_hbm.at[c, pl.ds(s * 32, 32), :])

    return seg_kernel(x, b3d)


def _onehot_kernel(batch_ref, x_ref, out_ref):
    i = pl.program_id(0)

    @pl.when(i == 0)
    def _init():
        out_ref[...] = jnp.zeros_like(out_ref)

    bvals = batch_ref[0, 0, :]
    xs = x_ref[...].astype(jnp.bfloat16)
    seg_ids = lax.broadcasted_iota(jnp.int32, (TC_BLK, NUM_SEGS), 1)
    onehot = (seg_ids == bvals[:, None]).astype(jnp.bfloat16)
    out_ref[...] += lax.dot_general(onehot, xs, (((0,), (0,)), ((), ())),
                                    preferred_element_type=jnp.float32)


def _mlp_kernel(p_ref, ptc_ref, w0_ref, b0_ref, w1_ref, b1_ref, w2_ref,
                b2_ref, out_ref):
    seg = p_ref[0] + p_ref[1] + ptc_ref[...]
    h0 = lax.dot_general(seg, w0_ref[...], (((1,), (1,)), ((), ())),
                         preferred_element_type=jnp.float32)
    h0 = jnp.maximum(h0 + b0_ref[...], 0.0)
    h1 = lax.dot_general(h0, w1_ref[...], (((1,), (1,)), ((), ())),
                         preferred_element_type=jnp.float32)
    h1 = jnp.maximum(h1 + b1_ref[...], 0.0)
    h2 = lax.dot_general(h1, w2_ref[...], (((1,), (1,)), ((), ())),
                         preferred_element_type=jnp.float32)
    out_ref[...] = h2 + b2_ref[...]


def kernel(x, batch, y, W0, b0, W1, b1, W2, b2):
    batch = batch.astype(jnp.int32)
    b3d = batch[TC_ROWS:].reshape(NW, NCH, CHUNK)
    partials = _sc_segment_sum(x, b3d)

    batch3 = batch[:TC_ROWS].reshape(TC_NBLK, 1, TC_BLK)
    partial_tc = pl.pallas_call(
        _onehot_kernel,
        grid=(TC_NBLK,),
        in_specs=[
            pl.BlockSpec((1, 1, TC_BLK), lambda i: (i, 0, 0)),
            pl.BlockSpec((TC_BLK, DIM), lambda i: (i, 0)),
        ],
        out_specs=pl.BlockSpec((NUM_SEGS, DIM), lambda i: (0, 0)),
        out_shape=jax.ShapeDtypeStruct((NUM_SEGS, DIM), jnp.float32),
    )(batch3, x)

    pred = pl.pallas_call(
        _mlp_kernel,
        in_specs=[
            pl.BlockSpec((2, NUM_SEGS, DIM), lambda: (0, 0, 0)),
            pl.BlockSpec((NUM_SEGS, DIM), lambda: (0, 0)),
            pl.BlockSpec((64, DIM), lambda: (0, 0)),
            pl.BlockSpec((1, 64), lambda: (0, 0)),
            pl.BlockSpec((32, 64), lambda: (0, 0)),
            pl.BlockSpec((1, 32), lambda: (0, 0)),
            pl.BlockSpec((8, 32), lambda: (0, 0)),
            pl.BlockSpec((1, 8), lambda: (0, 0)),
        ],
        out_specs=pl.BlockSpec((NUM_SEGS, 8), lambda: (0, 0)),
        out_shape=jax.ShapeDtypeStruct((NUM_SEGS, 8), jnp.float32),
    )(partials, partial_tc, W0, b0.reshape(1, 64), W1, b1.reshape(1, 32),
      jnp.pad(W2, ((0, 7), (0, 0))), jnp.pad(b2.reshape(1, 1), ((0, 0), (0, 7))))
    return (pred[:, :1], y)


# speedup vs baseline: 1.8444x; 1.8444x over previous
import functools

import jax
import jax.numpy as jnp
from jax import lax
from jax.experimental import pallas as pl
from jax.experimental.pallas import tpu as pltpu
from jax.experimental.pallas import tpu_sc as plsc

NUM_SEGS = 512
ROWS = 100000
DIM = 128
NQ = DIM // 16

TC_BLK = 2184
TC_NBLK = 20
TC_ROWS = TC_BLK * TC_NBLK

NW = 32
CHUNK = 32
NCH = 55
RPW = NCH * CHUNK
SC_ROWS = NW * RPW
assert TC_ROWS + SC_ROWS == ROWS
NBUF = 4


def _sc_segment_sum(x, b3d):
    mesh = plsc.VectorSubcoreMesh(core_axis_name="c", subcore_axis_name="s")

    @functools.partial(
        pl.kernel,
        out_type=jax.ShapeDtypeStruct((2, NUM_SEGS, DIM), jnp.float32),
        mesh=mesh,
        compiler_params=pltpu.CompilerParams(needs_layout_passes=False),
        scratch_types=[
            pltpu.VMEM((NBUF, CHUNK, DIM), jnp.float32),
            pltpu.VMEM((NCH, CHUNK), jnp.int32),
            pltpu.VMEM((CHUNK, DIM), jnp.float32),
            pltpu.VMEM_SHARED((NUM_SEGS, DIM), jnp.float32),
            pltpu.SemaphoreType.DMA,
            pltpu.SemaphoreType.DMA,
            pltpu.SemaphoreType.DMA,
            pltpu.SemaphoreType.DMA,
            pltpu.SemaphoreType.DMA,
            pltpu.SemaphoreType.DMA,
            pltpu.SemaphoreType.DMA,
            pltpu.SemaphoreType.DMA,
        ],
    )
    def seg_kernel(x_hbm, b_hbm, out_hbm, xbuf, bbuf, zbuf, shacc,
                   sx0, sx1, sx2, sx3, ss0, ss1, ss2, ss3):
        c = lax.axis_index("c")
        s = lax.axis_index("s")
        wid = s * 2 + c
        row0 = TC_ROWS + wid * RPW
        sx = [sx0, sx1, sx2, sx3]
        ss = [ss0, ss1, ss2, ss3]

        def x_copy(k, b):
            return pltpu.make_async_copy(
                x_hbm.at[pl.ds(row0 + k * CHUNK, CHUNK), :],
                xbuf.at[b], sx[b])

        def sc_copy(k, b):
            return pltpu.make_async_copy(
                xbuf.at[b], shacc.at[bbuf.at[k]], ss[b])

        for b in range(2):
            x_copy(b, b).start()

        pltpu.sync_copy(b_hbm.at[wid], bbuf)

        zeros16 = jnp.zeros((16,), jnp.float32)

        def _zrow(r, carry):
            for q in range(NQ):
                zbuf[r, pl.ds(16 * q, 16)] = zeros16
            return carry

        lax.fori_loop(0, CHUNK, _zrow, 0)
        pltpu.sync_copy(zbuf, shacc.at[pl.ds(s * 32, 32), :])
        plsc.subcore_barrier()

        def _chunk_body(i, carry):
            for b in range(NBUF):
                k = NBUF * i + b

                @pl.when(k < NCH)
                def _work():
                    x_copy(k, b).wait()
                    sc_copy(k, b).start(add=True)

                if b < 2:
                    @pl.when(jnp.logical_and(k >= 2, k - 2 < NCH))
                    def _drain():
                        sc_copy(k - 2, (b + 2) % NBUF).wait()
                else:
                    @pl.when(k - 2 < NCH)
                    def _drain2():
                        sc_copy(k - 2, (b + 2) % NBUF).wait()

                @pl.when(k + 2 < NCH)
                def _prefetch():
                    x_copy(k + 2, (b + 2) % NBUF).start()

            return carry

        lax.fori_loop(0, (NCH + 2 + NBUF - 1) // NBUF, _chunk_body, 0)
        plsc.subcore_barrier()

        pltpu.sync_copy(shacc.at[pl.ds(s * 32, 32), :],
                        out_hbm.at[c, pl.ds(s * 32, 32), :])

    return seg_kernel(x, b3d)


def _onehot_kernel(batch_ref, x_ref, out_ref):
    i = pl.program_id(0)

    @pl.when(i == 0)
    def _init():
        out_ref[...] = jnp.zeros_like(out_ref)

    bvals = batch_ref[0, 0, :]
    xs = x_ref[...]
    seg_ids = lax.broadcasted_iota(jnp.int32, (TC_BLK, NUM_SEGS), 1)
    onehot = (seg_ids == bvals[:, None]).astype(jnp.float32)
    out_ref[...] += lax.dot_general(onehot, xs, (((0,), (0,)), ((), ())),
                                    preferred_element_type=jnp.float32)


def _mlp_kernel(p_ref, ptc_ref, w0_ref, b0_ref, w1_ref, b1_ref, w2_ref,
                b2_ref, out_ref):
    seg = p_ref[0] + p_ref[1] + ptc_ref[...]
    h0 = lax.dot_general(seg, w0_ref[...], (((1,), (1,)), ((), ())),
                         preferred_element_type=jnp.float32)
    h0 = jnp.maximum(h0 + b0_ref[...], 0.0)
    h1 = lax.dot_general(h0, w1_ref[...], (((1,), (1,)), ((), ())),
                         preferred_element_type=jnp.float32)
    h1 = jnp.maximum(h1 + b1_ref[...], 0.0)
    h2 = lax.dot_general(h1, w2_ref[...], (((1,), (1,)), ((), ())),
                         preferred_element_type=jnp.float32)
    out_ref[...] = h2 + b2_ref[...]


def kernel(x, batch, y, W0, b0, W1, b1, W2, b2):
    batch = batch.astype(jnp.int32)
    b3d = batch[TC_ROWS:].reshape(NW, NCH, CHUNK)
    partials = _sc_segment_sum(x, b3d)

    batch3 = batch[:TC_ROWS].reshape(TC_NBLK, 1, TC_BLK)
    partial_tc = pl.pallas_call(
        _onehot_kernel,
        grid=(TC_NBLK,),
        in_specs=[
            pl.BlockSpec((1, 1, TC_BLK), lambda i: (i, 0, 0)),
            pl.BlockSpec((TC_BLK, DIM), lambda i: (i, 0)),
        ],
        out_specs=pl.BlockSpec((NUM_SEGS, DIM), lambda i: (0, 0)),
        out_shape=jax.ShapeDtypeStruct((NUM_SEGS, DIM), jnp.float32),
    )(batch3, x)

    pred = pl.pallas_call(
        _mlp_kernel,
        in_specs=[
            pl.BlockSpec((2, NUM_SEGS, DIM), lambda: (0, 0, 0)),
            pl.BlockSpec((NUM_SEGS, DIM), lambda: (0, 0)),
            pl.BlockSpec((64, DIM), lambda: (0, 0)),
            pl.BlockSpec((1, 64), lambda: (0, 0)),
            pl.BlockSpec((32, 64), lambda: (0, 0)),
            pl.BlockSpec((1, 32), lambda: (0, 0)),
            pl.BlockSpec((8, 32), lambda: (0, 0)),
            pl.BlockSpec((1, 8), lambda: (0, 0)),
        ],
        out_specs=pl.BlockSpec((NUM_SEGS, 8), lambda: (0, 0)),
        out_shape=jax.ShapeDtypeStruct((NUM_SEGS, 8), jnp.float32),
    )(partials, partial_tc, W0, b0.reshape(1, 64), W1, b1.reshape(1, 32),
      jnp.pad(W2, ((0, 7), (0, 0))), jnp.pad(b2.reshape(1, 1), ((0, 0), (0, 7))))
    return (pred[:, :1], y)
